# baseline (speedup 1.0000x reference)
import functools

import jax
import jax.numpy as jnp
from jax.experimental import pallas as pl
from jax.experimental.pallas import tpu as pltpu

_C_IN = 32
_C_OUT = 192
_BN_EPS = 1e-5
_N_CORES = 2
_BLOCK_BATCH = 2
_OUT_RING = 4


def _stats_sum_kernel(x71_ref, x57_ref, xs_ref, s_ref, g_ref, *, bb):
    step = pl.program_id(1)
    x = x71_ref[...] + x57_ref[...]
    xs_ref[...] = x.astype(jnp.bfloat16)

    @pl.when(step == 0)
    def _init():
        s_ref[...] = jnp.zeros_like(s_ref)
        g_ref[...] = jnp.zeros_like(g_ref)

    s_ref[0] += jnp.sum(x, axis=(0, 2))[:, None]
    g = jnp.zeros((_C_IN, _C_IN), jnp.float32)
    for b in range(bb):
        g = g + jax.lax.dot_general(
            x[b], x[b], (((1,), (1,)), ((), ())),
            preferred_element_type=jnp.float32)
    g_ref[0] += g


def _fold_conv_kernel(xs_ref, s_ref, g_ref, w_ref, gamma_ref, beta_ref,
                      o_hbm, wf_ref, bf_ref, obuf, osem, *, count, bb, steps,
                      ring):
    ci = pl.program_id(0)
    step = pl.program_id(1)

    @pl.when(step == 0)
    def _fold():
        inv = 1.0 / count
        mean_x = (s_ref[0] + s_ref[1]) * inv
        exx = (g_ref[0] + g_ref[1]) * inv
        w = w_ref[...]
        mean_y = jnp.dot(w, mean_x, preferred_element_type=jnp.float32)
        e_y2 = jnp.sum(jnp.dot(w, exx, preferred_element_type=jnp.float32) * w,
                       axis=1, keepdims=True)
        var_y = jnp.maximum(e_y2 - mean_y * mean_y, 0.0)
        scale = gamma_ref[...] * jax.lax.rsqrt(var_y + _BN_EPS)
        wf_ref[...] = (w * scale).astype(jnp.bfloat16)
        bf_ref[...] = beta_ref[...] - mean_y * scale

    slot = jax.lax.rem(step, ring)

    @pl.when(step >= ring)
    def _reuse_wait():
        pltpu.make_async_copy(obuf.at[slot], o_hbm.at[pl.ds(0, bb)],
                              osem.at[slot]).wait()

    wf = wf_ref[...]
    bias = bf_ref[...]
    for b in range(bb):
        y = jnp.dot(wf, xs_ref[b],
                    preferred_element_type=jnp.float32)
        obuf[slot, b] = y + bias

    base = (ci * steps + step) * bb
    pltpu.make_async_copy(obuf.at[slot], o_hbm.at[pl.ds(base, bb)],
                          osem.at[slot]).start()

    @pl.when(step == steps - 1)
    def _drain():
        for k in range(min(ring, steps)):
            pltpu.make_async_copy(obuf.at[k], o_hbm.at[pl.ds(0, bb)],
                                  osem.at[k]).wait()


def kernel(x71, x57, weight, gamma, beta):
    n, c, h, w = x71.shape
    assert c == _C_IN and x57.shape == x71.shape and n % _N_CORES == 0
    hw = h * w
    per_core = n // _N_CORES
    bb = _BLOCK_BATCH if per_core % _BLOCK_BATCH == 0 else 1
    steps = per_core // bb

    x71_r = x71.reshape(n, _C_IN, hw)
    x57_r = x57.reshape(n, _C_IN, hw)
    w_mat = weight.astype(jnp.float32).reshape(_C_OUT, _C_IN)
    g_col = gamma.astype(jnp.float32).reshape(_C_OUT, 1)
    b_col = beta.astype(jnp.float32).reshape(_C_OUT, 1)

    batch_map = lambda ci, bi: (ci * steps + bi, 0, 0)

    xs, s_part, g_part = pl.pallas_call(
        functools.partial(_stats_sum_kernel, bb=bb),
        out_shape=(
            jax.ShapeDtypeStruct((n, _C_IN, hw), jnp.bfloat16),
            jax.ShapeDtypeStruct((_N_CORES, _C_IN, 1), jnp.float32),
            jax.ShapeDtypeStruct((_N_CORES, _C_IN, _C_IN), jnp.float32),
        ),
        grid=(_N_CORES, steps),
        in_specs=[
            pl.BlockSpec((bb, _C_IN, hw), batch_map),
            pl.BlockSpec((bb, _C_IN, hw), batch_map),
        ],
        out_specs=(
            pl.BlockSpec((bb, _C_IN, hw), batch_map),
            pl.BlockSpec((1, _C_IN, 1), lambda ci, bi: (ci, 0, 0)),
            pl.BlockSpec((1, _C_IN, _C_IN), lambda ci, bi: (ci, 0, 0)),
        ),
        compiler_params=pltpu.CompilerParams(
            dimension_semantics=("parallel", "arbitrary")),
    )(x71_r, x57_r)

    ring = min(_OUT_RING, steps)
    out = pl.pallas_call(
        functools.partial(_fold_conv_kernel, count=float(n * hw), bb=bb,
                          steps=steps, ring=ring),
        out_shape=jax.ShapeDtypeStruct((n, _C_OUT, hw), jnp.float32),
        grid=(_N_CORES, steps),
        in_specs=[
            pl.BlockSpec((bb, _C_IN, hw), batch_map),
            pl.BlockSpec((_N_CORES, _C_IN, 1), lambda ci, bi: (0, 0, 0)),
            pl.BlockSpec((_N_CORES, _C_IN, _C_IN), lambda ci, bi: (0, 0, 0)),
            pl.BlockSpec((_C_OUT, _C_IN), lambda ci, bi: (0, 0)),
            pl.BlockSpec((_C_OUT, 1), lambda ci, bi: (0, 0)),
            pl.BlockSpec((_C_OUT, 1), lambda ci, bi: (0, 0)),
        ],
        out_specs=pl.BlockSpec(memory_space=pl.ANY),
        scratch_shapes=[
            pltpu.VMEM((_C_OUT, _C_IN), jnp.bfloat16),
            pltpu.VMEM((_C_OUT, 1), jnp.float32),
            pltpu.VMEM((ring, bb, _C_OUT, hw), jnp.float32),
            pltpu.SemaphoreType.DMA((ring,)),
        ],
        compiler_params=pltpu.CompilerParams(
            dimension_semantics=("parallel", "arbitrary")),
    )(xs, s_part, g_part, w_mat, g_col, b_col)

    return out.reshape(n, _C_OUT, h, w)


# speedup vs baseline: 1.0653x; 1.0653x over previous
import functools

import jax
import jax.numpy as jnp
from jax.experimental import pallas as pl
from jax.experimental.pallas import tpu as pltpu

_C_IN = 32
_C_OUT = 192
_BN_EPS = 1e-5
_N_CORES = 2
_BLOCK_BATCH = 4


def _stats_sum_kernel(x71_ref, x57_ref, xs_ref, s_ref, g_ref, *, bb):
    step = pl.program_id(1)
    x = x71_ref[...] + x57_ref[...]
    xs_ref[...] = x.astype(jnp.bfloat16)

    @pl.when(step == 0)
    def _init():
        s_ref[...] = jnp.zeros_like(s_ref)
        g_ref[...] = jnp.zeros_like(g_ref)

    s_ref[0] += jnp.sum(x, axis=(0, 2))[:, None]
    g = jnp.zeros((_C_IN, _C_IN), jnp.float32)
    for b in range(bb):
        g = g + jax.lax.dot_general(
            x[b], x[b], (((1,), (1,)), ((), ())),
            preferred_element_type=jnp.float32)
    g_ref[0] += g


def _fold_conv_kernel(xs_ref, s_ref, g_ref, w_ref, gamma_ref, beta_ref,
                      o_ref, wf_ref, bf_ref, *, count, bb):
    step = pl.program_id(1)

    @pl.when(step == 0)
    def _fold():
        inv = 1.0 / count
        mean_x = (s_ref[0] + s_ref[1]) * inv
        exx = (g_ref[0] + g_ref[1]) * inv
        w = w_ref[...]
        mean_y = jnp.dot(w, mean_x, preferred_element_type=jnp.float32)
        e_y2 = jnp.sum(jnp.dot(w, exx, preferred_element_type=jnp.float32) * w,
                       axis=1, keepdims=True)
        var_y = jnp.maximum(e_y2 - mean_y * mean_y, 0.0)
        scale = gamma_ref[...] * jax.lax.rsqrt(var_y + _BN_EPS)
        wf_ref[...] = (w * scale).astype(jnp.bfloat16)
        bf_ref[...] = beta_ref[...] - mean_y * scale

    wf = wf_ref[...]
    bias = bf_ref[...]
    for b in range(bb):
        y = jnp.dot(wf, xs_ref[b],
                    preferred_element_type=jnp.float32)
        o_ref[b] = y + bias


def kernel(x71, x57, weight, gamma, beta):
    n, c, h, w = x71.shape
    assert c == _C_IN and x57.shape == x71.shape and n % _N_CORES == 0
    hw = h * w
    per_core = n // _N_CORES
    bb = _BLOCK_BATCH if per_core % _BLOCK_BATCH == 0 else 1
    steps = per_core // bb

    x71_r = x71.reshape(n, _C_IN, hw)
    x57_r = x57.reshape(n, _C_IN, hw)
    w_mat = weight.astype(jnp.float32).reshape(_C_OUT, _C_IN)
    g_col = gamma.astype(jnp.float32).reshape(_C_OUT, 1)
    b_col = beta.astype(jnp.float32).reshape(_C_OUT, 1)

    batch_map = lambda ci, bi: (ci * steps + bi, 0, 0)

    xs, s_part, g_part = pl.pallas_call(
        functools.partial(_stats_sum_kernel, bb=bb),
        out_shape=(
            jax.ShapeDtypeStruct((n, _C_IN, hw), jnp.bfloat16),
            jax.ShapeDtypeStruct((_N_CORES, _C_IN, 1), jnp.float32),
            jax.ShapeDtypeStruct((_N_CORES, _C_IN, _C_IN), jnp.float32),
        ),
        grid=(_N_CORES, steps),
        in_specs=[
            pl.BlockSpec((bb, _C_IN, hw), batch_map),
            pl.BlockSpec((bb, _C_IN, hw), batch_map),
        ],
        out_specs=(
            pl.BlockSpec((bb, _C_IN, hw), batch_map),
            pl.BlockSpec((1, _C_IN, 1), lambda ci, bi: (ci, 0, 0)),
            pl.BlockSpec((1, _C_IN, _C_IN), lambda ci, bi: (ci, 0, 0)),
        ),
        compiler_params=pltpu.CompilerParams(
            dimension_semantics=("parallel", "arbitrary"),
            vmem_limit_bytes=24 * 1024 * 1024),
    )(x71_r, x57_r)

    out = pl.pallas_call(
        functools.partial(_fold_conv_kernel, count=float(n * hw), bb=bb),
        out_shape=jax.ShapeDtypeStruct((n, _C_OUT, hw), jnp.float32),
        grid=(_N_CORES, steps),
        in_specs=[
            pl.BlockSpec((bb, _C_IN, hw), batch_map),
            pl.BlockSpec((_N_CORES, _C_IN, 1), lambda ci, bi: (0, 0, 0)),
            pl.BlockSpec((_N_CORES, _C_IN, _C_IN), lambda ci, bi: (0, 0, 0)),
            pl.BlockSpec((_C_OUT, _C_IN), lambda ci, bi: (0, 0)),
            pl.BlockSpec((_C_OUT, 1), lambda ci, bi: (0, 0)),
            pl.BlockSpec((_C_OUT, 1), lambda ci, bi: (0, 0)),
        ],
        out_specs=pl.BlockSpec((bb, _C_OUT, hw), batch_map),
        scratch_shapes=[
            pltpu.VMEM((_C_OUT, _C_IN), jnp.bfloat16),
            pltpu.VMEM((_C_OUT, 1), jnp.float32),
        ],
        compiler_params=pltpu.CompilerParams(
            dimension_semantics=("parallel", "arbitrary"),
            vmem_limit_bytes=26 * 1024 * 1024),
    )(xs, s_part, g_part, w_mat, g_col, b_col)

    return out.reshape(n, _C_OUT, h, w)
